# Initial kernel scaffold; baseline (speedup 1.0000x reference)
import jax
import jax.numpy as jnp
from jax.experimental import pallas as pl


def kernel(x, edge_index, W_in, b_in, Wb, bb, W1, b1, W2, b2, W3, b3):
    raise NotImplementedError("write your pallas kernel here")


# speedup vs baseline: 8.4901x; 8.4901x over previous
import jax
import jax.numpy as jnp
from jax.experimental import pallas as pl

N = 512
E = 16384
E_TILE = 2048
N_EDGE_TILES = E // E_TILE


def _adj_kernel(src_ref, dst_ref, c_ref):
    i = pl.program_id(0)
    s = src_ref[0, 0, :]
    d = dst_ref[0, 0, :]
    row_ids = jax.lax.broadcasted_iota(jnp.int32, (N, E_TILE), 0)
    col_ids = jax.lax.broadcasted_iota(jnp.int32, (E_TILE, N), 1)
    oh_d = (row_ids == d[None, :]).astype(jnp.float32)
    oh_s = (col_ids == s[:, None]).astype(jnp.float32)
    contrib = jnp.dot(oh_d, oh_s, preferred_element_type=jnp.float32)

    @pl.when(i == 0)
    def _():
        eye = (jax.lax.broadcasted_iota(jnp.int32, (N, N), 0)
               == jax.lax.broadcasted_iota(jnp.int32, (N, N), 1))
        c_ref[...] = eye.astype(jnp.float32) + contrib

    @pl.when(i > 0)
    def _():
        c_ref[...] += contrib


def _gcn_kernel(c_ref, x_ref, win_ref, bin_ref, wb_ref, bb_ref, h_ref):
    C = c_ref[...]
    deg = jnp.sum(C, axis=1)
    dinv = jax.lax.rsqrt(jnp.maximum(deg, 1e-12))
    A = C * dinv[:, None] * dinv[None, :]

    def conv(h, W, b):
        hw = jnp.dot(h, W, preferred_element_type=jnp.float32)
        return jnp.dot(A, hw, preferred_element_type=jnp.float32) + b

    h = jnp.maximum(conv(x_ref[...], win_ref[...], bin_ref[0, :]), 0.0)
    for i in range(3):
        t = jnp.maximum(conv(h, wb_ref[2 * i], bb_ref[2 * i]), 0.0)
        t = conv(t, wb_ref[2 * i + 1], bb_ref[2 * i + 1])
        h = jnp.maximum(t + h, 0.0)
    h_ref[...] = h


def _mlp1_kernel(v_ref, w_ref, b_ref, o_ref):
    k = pl.program_id(1)

    @pl.when(k == 0)
    def _():
        o_ref[...] = jnp.zeros_like(o_ref)

    o_ref[...] += jnp.dot(v_ref[...], w_ref[...],
                          preferred_element_type=jnp.float32)

    @pl.when(k == pl.num_programs(1) - 1)
    def _():
        o_ref[...] = jnp.maximum(o_ref[...] + b_ref[...], 0.0)


def _mlp2_kernel(v_ref, w_ref, b_ref, o_ref):
    o_ref[...] = jnp.maximum(
        jnp.dot(v_ref[...], w_ref[...], preferred_element_type=jnp.float32)
        + b_ref[...], 0.0)


def _mlp3_kernel(v_ref, w_ref, b_ref, o_ref):
    logits = (jnp.dot(v_ref[...], w_ref[...],
                      preferred_element_type=jnp.float32) + b_ref[...])
    m = jnp.max(logits, axis=-1, keepdims=True)
    e = jnp.exp(logits - m)
    o_ref[...] = e / jnp.sum(e, axis=-1, keepdims=True)


def kernel(x, edge_index, W_in, b_in, Wb, bb, W1, b1, W2, b2, W3, b3):
    src = edge_index[0].reshape(N_EDGE_TILES, 1, E_TILE)
    dst = edge_index[1].reshape(N_EDGE_TILES, 1, E_TILE)

    C = pl.pallas_call(
        _adj_kernel,
        grid=(N_EDGE_TILES,),
        in_specs=[
            pl.BlockSpec((1, 1, E_TILE), lambda i: (i, 0, 0)),
            pl.BlockSpec((1, 1, E_TILE), lambda i: (i, 0, 0)),
        ],
        out_specs=pl.BlockSpec((N, N), lambda i: (0, 0)),
        out_shape=jax.ShapeDtypeStruct((N, N), jnp.float32),
    )(src, dst)

    h = pl.pallas_call(
        _gcn_kernel,
        in_specs=[
            pl.BlockSpec((N, N), lambda: (0, 0)),
            pl.BlockSpec((N, 64), lambda: (0, 0)),
            pl.BlockSpec((64, 32), lambda: (0, 0)),
            pl.BlockSpec((1, 32), lambda: (0, 0)),
            pl.BlockSpec((6, 32, 32), lambda: (0, 0, 0)),
            pl.BlockSpec((6, 32), lambda: (0, 0)),
        ],
        out_specs=pl.BlockSpec((N, 32), lambda: (0, 0)),
        out_shape=jax.ShapeDtypeStruct((N, 32), jnp.float32),
    )(C, x, W_in, b_in.reshape(1, 32), Wb.reshape(6, 32, 32),
      bb.reshape(6, 32))

    v = h.reshape(1, N * 32)

    KT, CT = 2048, 2048
    v1 = pl.pallas_call(
        _mlp1_kernel,
        grid=(W1.shape[1] // CT, W1.shape[0] // KT),
        in_specs=[
            pl.BlockSpec((1, KT), lambda c, k: (0, k)),
            pl.BlockSpec((KT, CT), lambda c, k: (k, c)),
            pl.BlockSpec((1, CT), lambda c, k: (0, c)),
        ],
        out_specs=pl.BlockSpec((1, CT), lambda c, k: (0, c)),
        out_shape=jax.ShapeDtypeStruct((1, W1.shape[1]), jnp.float32),
    )(v, W1, b1.reshape(1, -1))

    C2 = 1024
    v2 = pl.pallas_call(
        _mlp2_kernel,
        grid=(W2.shape[1] // C2,),
        in_specs=[
            pl.BlockSpec((1, W2.shape[0]), lambda c: (0, 0)),
            pl.BlockSpec((W2.shape[0], C2), lambda c: (0, c)),
            pl.BlockSpec((1, C2), lambda c: (0, c)),
        ],
        out_specs=pl.BlockSpec((1, C2), lambda c: (0, c)),
        out_shape=jax.ShapeDtypeStruct((1, W2.shape[1]), jnp.float32),
    )(v1, W2, b2.reshape(1, -1))

    out = pl.pallas_call(
        _mlp3_kernel,
        in_specs=[
            pl.BlockSpec((1, W3.shape[0]), lambda: (0, 0)),
            pl.BlockSpec((W3.shape[0], 10), lambda: (0, 0)),
            pl.BlockSpec((1, 10), lambda: (0, 0)),
        ],
        out_specs=pl.BlockSpec((1, 10), lambda: (0, 0)),
        out_shape=jax.ShapeDtypeStruct((1, 10), jnp.float32),
    )(v2, W3, b3.reshape(1, -1))

    return out.reshape(10)
